# baseline (speedup 1.0000x reference)
import functools

import jax
import jax.numpy as jnp
from jax import lax
from jax.experimental import pallas as pl
from jax.experimental.pallas import tpu as pltpu
from jax.experimental.pallas import tpu_sc as plsc

_GAMMA = 12.0
_HID = 128
_LANES = 16
_NSUB = _HID // _LANES
_NC, _NS = 2, 16
_NW = _NC * _NS
_CHUNK = 64
_NBUF = 4
_GPC = _CHUNK // _LANES


def _tree_sum(vs):
  while len(vs) > 1:
    vs = [a + b for a, b in zip(vs[::2], vs[1::2])] + (
        [vs[-1]] if len(vs) % 2 else [])
  return vs[0]


def _make_sc_call(batch):
  bw = batch // _NW
  nchunk = bw // _CHUNK
  ngroups = bw // _LANES

  mesh = plsc.VectorSubcoreMesh(core_axis_name="c", subcore_axis_name="s")

  @functools.partial(
      pl.kernel,
      out_type=jax.ShapeDtypeStruct((batch,), jnp.float32),
      mesh=mesh,
      compiler_params=pltpu.CompilerParams(needs_layout_passes=False),
      scratch_types=[
          pltpu.VMEM((nchunk, _CHUNK), jnp.int32),
          pltpu.VMEM((nchunk, _CHUNK), jnp.int32),
          pltpu.VMEM((nchunk, _CHUNK), jnp.int32),
          pltpu.VMEM((_NBUF, _CHUNK, _HID), jnp.float32),
          pltpu.VMEM((_NBUF, _CHUNK, _HID), jnp.float32),
          pltpu.VMEM((_NBUF, _CHUNK, _HID), jnp.float32),
          pltpu.VMEM((bw,), jnp.float32),
          pltpu.VMEM((_LANES * (_LANES + 1),), jnp.float32),
          pltpu.SemaphoreType.DMA((_NBUF,)),
          pltpu.SemaphoreType.DMA((_NBUF,)),
          pltpu.SemaphoreType.DMA((_NBUF,)),
      ],
  )
  def sc_score(ent_hbm, rel_hbm, hidx_hbm, ridx_hbm, tidx_hbm, out_hbm,
               hidx, ridx, tidx, hb, rb, tb, ob, tsc,
               hsem, rsem, tsem):
    wid = lax.axis_index("s") * _NC + lax.axis_index("c")
    base = wid * bw

    row0 = wid * nchunk
    idx_cps = [
        pltpu.async_copy(hidx_hbm.at[pl.ds(row0, nchunk), :], hidx,
                         hsem.at[0]),
        pltpu.async_copy(ridx_hbm.at[pl.ds(row0, nchunk), :], ridx,
                         rsem.at[0]),
        pltpu.async_copy(tidx_hbm.at[pl.ds(row0, nchunk), :], tidx,
                         tsem.at[0]),
    ]
    for cp in idx_cps:
      cp.wait()

    def issue(c, slot):
      pltpu.async_copy(ent_hbm.at[hidx.at[c]], hb.at[slot], hsem.at[slot])
      pltpu.async_copy(rel_hbm.at[ridx.at[c]], rb.at[slot], rsem.at[slot])
      pltpu.async_copy(ent_hbm.at[tidx.at[c]], tb.at[slot], tsem.at[slot])

    def wait(c, slot):
      pltpu.make_async_copy(
          ent_hbm.at[hidx.at[c]], hb.at[slot], hsem.at[slot]).wait()
      pltpu.make_async_copy(
          rel_hbm.at[ridx.at[c]], rb.at[slot], rsem.at[slot]).wait()
      pltpu.make_async_copy(
          ent_hbm.at[tidx.at[c]], tb.at[slot], tsem.at[slot]).wait()

    for c in range(min(_NBUF - 1, nchunk)):
      issue(c, c)

    col_rows = lax.iota(jnp.int32, _LANES) * (_LANES + 1)
    gamma_v = jnp.full((_LANES,), _GAMMA, jnp.float32)

    def group_body(g, _):
      c = g // _GPC
      slot = lax.rem(c, _NBUF)

      @pl.when(lax.rem(g, _GPC) == 0)
      def _chunk_edge():
        wait(c, slot)
        nc = c + _NBUF - 1

        @pl.when(nc < nchunk)
        def _prefetch():
          issue(nc, lax.rem(nc, _NBUF))

      s0 = lax.rem(g, _GPC) * _LANES
      for i in range(_LANES):
        s = s0 + i
        vs = []
        for d in range(_NSUB):
          f = pl.ds(d * _LANES, _LANES)
          vs.append(jnp.abs(hb[slot, s, f] + rb[slot, s, f] - tb[slot, s, f]))
        tsc[pl.ds(i * (_LANES + 1), _LANES)] = _tree_sum(vs)
      tot = _tree_sum(
          [plsc.load_gather(tsc, [col_rows + j]) for j in range(_LANES)])
      ob[pl.ds(g * _LANES, _LANES)] = gamma_v - tot
      return ()

    lax.fori_loop(0, ngroups, group_body, (), unroll=False)

    pltpu.sync_copy(ob, out_hbm.at[pl.ds(base, bw)])

  return sc_score


def kernel(entity_embedding, relation_embedding, sample):
  batch = sample.shape[0]
  scores = _make_sc_call(batch)(
      entity_embedding, relation_embedding,
      sample[:, 0].reshape(-1, _CHUNK),
      sample[:, 1].reshape(-1, _CHUNK),
      sample[:, 2].reshape(-1, _CHUNK))
  return scores[:, None]


# speedup vs baseline: 1.0203x; 1.0203x over previous
import functools

import jax
import jax.numpy as jnp
from jax import lax
from jax.experimental import pallas as pl
from jax.experimental.pallas import tpu as pltpu
from jax.experimental.pallas import tpu_sc as plsc

_GAMMA = 12.0
_HID = 128
_LANES = 16
_NSUB = _HID // _LANES
_NC, _NS = 2, 16
_NW = _NC * _NS
_CHUNK = 64
_NBUF = 4
_GPC = _CHUNK // _LANES


def _make_sc_call(batch):
  bw = batch // _NW
  nchunk = bw // _CHUNK
  ngroups = bw // _LANES

  mesh = plsc.VectorSubcoreMesh(core_axis_name="c", subcore_axis_name="s")

  @functools.partial(
      pl.kernel,
      out_type=jax.ShapeDtypeStruct((batch,), jnp.float32),
      mesh=mesh,
      compiler_params=pltpu.CompilerParams(needs_layout_passes=False),
      scratch_types=[
          pltpu.VMEM((nchunk, _CHUNK), jnp.int32),
          pltpu.VMEM((nchunk, _CHUNK), jnp.int32),
          pltpu.VMEM((nchunk, _CHUNK), jnp.int32),
          pltpu.VMEM((_NBUF, _CHUNK, _HID), jnp.float32),
          pltpu.VMEM((_NBUF, _CHUNK, _HID), jnp.float32),
          pltpu.VMEM((_NBUF, _CHUNK, _HID), jnp.float32),
          pltpu.VMEM((bw,), jnp.float32),
          pltpu.VMEM((_LANES * (_LANES + 1),), jnp.float32),
          pltpu.SemaphoreType.DMA((_NBUF,)),
          pltpu.SemaphoreType.DMA((_NBUF,)),
          pltpu.SemaphoreType.DMA((_NBUF,)),
      ],
  )
  def sc_score(ent_hbm, rel_hbm, hidx_hbm, ridx_hbm, tidx_hbm, out_hbm,
               hidx, ridx, tidx, hb, rb, tb, ob, tsc,
               hsem, rsem, tsem):
    wid = lax.axis_index("s") * _NC + lax.axis_index("c")
    base = wid * bw

    row0 = wid * nchunk
    idx_cps = [
        pltpu.async_copy(hidx_hbm.at[pl.ds(row0, nchunk), :], hidx,
                         hsem.at[0]),
        pltpu.async_copy(ridx_hbm.at[pl.ds(row0, nchunk), :], ridx,
                         rsem.at[0]),
        pltpu.async_copy(tidx_hbm.at[pl.ds(row0, nchunk), :], tidx,
                         tsem.at[0]),
    ]
    for cp in idx_cps:
      cp.wait()

    def issue(c, slot):
      pltpu.async_copy(ent_hbm.at[hidx.at[c]], hb.at[slot], hsem.at[slot])
      pltpu.async_copy(rel_hbm.at[ridx.at[c]], rb.at[slot], rsem.at[slot])
      pltpu.async_copy(ent_hbm.at[tidx.at[c]], tb.at[slot], tsem.at[slot])

    def wait(c, slot):
      pltpu.make_async_copy(
          ent_hbm.at[hidx.at[c]], hb.at[slot], hsem.at[slot]).wait()
      pltpu.make_async_copy(
          rel_hbm.at[ridx.at[c]], rb.at[slot], rsem.at[slot]).wait()
      pltpu.make_async_copy(
          ent_hbm.at[tidx.at[c]], tb.at[slot], tsem.at[slot]).wait()

    for c in range(min(_NBUF - 1, nchunk)):
      issue(c, c)

    col_rows = lax.iota(jnp.int32, _LANES) * (_LANES + 1)
    gamma_v = jnp.full((_LANES,), _GAMMA, jnp.float32)

    def group_body(g, _):
      c = g // _GPC
      slot = lax.rem(c, _NBUF)

      @pl.when(lax.rem(g, _GPC) == 0)
      def _chunk_edge():
        wait(c, slot)
        nc = c + _NBUF - 1

        @pl.when(nc < nchunk)
        def _prefetch():
          issue(nc, lax.rem(nc, _NBUF))

      s0 = lax.rem(g, _GPC) * _LANES
      for i in range(_LANES):
        s = s0 + i
        acc = None
        for d in range(_NSUB):
          f = pl.ds(d * _LANES, _LANES)
          v = jnp.abs(hb[slot, s, f] + rb[slot, s, f] - tb[slot, s, f])
          acc = v if acc is None else acc + v
        tsc[pl.ds(i * (_LANES + 1), _LANES)] = acc
      tot = plsc.load_gather(tsc, [col_rows])
      for j in range(1, _LANES):
        tot = tot + plsc.load_gather(tsc, [col_rows + j])
      ob[pl.ds(g * _LANES, _LANES)] = gamma_v - tot
      return ()

    lax.fori_loop(0, ngroups, group_body, (), unroll=False)

    pltpu.sync_copy(ob, out_hbm.at[pl.ds(base, bw)])

  return sc_score


def kernel(entity_embedding, relation_embedding, sample):
  batch = sample.shape[0]
  scores = _make_sc_call(batch)(
      entity_embedding, relation_embedding,
      sample[:, 0].reshape(-1, _CHUNK),
      sample[:, 1].reshape(-1, _CHUNK),
      sample[:, 2].reshape(-1, _CHUNK))
  return scores[:, None]
